# baseline (speedup 1.0000x reference)
import functools
import jax
import jax.numpy as jnp
from jax.experimental import pallas as pl

_N = 50000
_HF = 64
_ROWS = 2000


def _mm_body(x_ref, w_ref, b_ref, o_ref, s_ref):
    i = pl.program_id(0)
    h = jnp.dot(x_ref[...], w_ref[...], preferred_element_type=jnp.float32)
    h = h + b_ref[...]
    o_ref[...] = h

    @pl.when(i == 0)
    def _():
        s_ref[...] = jnp.zeros_like(s_ref)

    s_ref[0:1, :] += jnp.sum(h, axis=0, keepdims=True)
    s_ref[1:2, :] += jnp.sum(h * h, axis=0, keepdims=True)


def _matmul_bias(x, w, b):
    n, k = x.shape
    m = w.shape[1]
    grid = n // _ROWS
    out, stats = pl.pallas_call(
        _mm_body,
        grid=(grid,),
        in_specs=[
            pl.BlockSpec((_ROWS, k), lambda i: (i, 0)),
            pl.BlockSpec((k, m), lambda i: (0, 0)),
            pl.BlockSpec((1, m), lambda i: (0, 0)),
        ],
        out_specs=[
            pl.BlockSpec((_ROWS, m), lambda i: (i, 0)),
            pl.BlockSpec((2, m), lambda i: (0, 0)),
        ],
        out_shape=[
            jax.ShapeDtypeStruct((n, m), jnp.float32),
            jax.ShapeDtypeStruct((2, m), jnp.float32),
        ],
    )(x, w, b.reshape(1, -1))
    return out, stats


def _bn_body(h_ref, s_ref, g_ref, b_ref, o_ref):
    s = s_ref[...]
    mean = s[0:1, :] / _N
    var = s[1:2, :] / _N - mean * mean
    y = (h_ref[...] - mean) * jax.lax.rsqrt(var + 1e-5) * g_ref[...] + b_ref[...]
    o_ref[...] = jnp.where(y > 0, y, 0.01 * y)


def _bn_leaky(h, stats, g, b):
    n, m = h.shape
    return pl.pallas_call(
        _bn_body,
        grid=(n // _ROWS,),
        in_specs=[
            pl.BlockSpec((_ROWS, m), lambda i: (i, 0)),
            pl.BlockSpec((2, m), lambda i: (0, 0)),
            pl.BlockSpec((1, m), lambda i: (0, 0)),
            pl.BlockSpec((1, m), lambda i: (0, 0)),
        ],
        out_specs=pl.BlockSpec((_ROWS, m), lambda i: (i, 0)),
        out_shape=jax.ShapeDtypeStruct((n, m), jnp.float32),
    )(h, stats, g.reshape(1, -1), b.reshape(1, -1))


def _gat_edge(feat, el, er, src, dst, n, heads, bias):
    e = jax.nn.leaky_relu(el[src] + er[dst], 0.2)
    emax = jax.ops.segment_max(e, dst, num_segments=n)
    emax = jnp.where(jnp.isfinite(emax), emax, 0.0)
    ee = jnp.exp(e - emax[dst])
    den = jax.ops.segment_sum(ee, dst, num_segments=n)
    alpha = ee / (den[dst] + 1e-9)
    msg = feat.reshape(n, heads, _HF)[src] * alpha[:, :, None]
    rst = jax.ops.segment_sum(msg, dst, num_segments=n)
    out = rst + bias.reshape(1, heads, _HF)
    out = jnp.where(out > 0, out, 0.01 * out)
    return out.reshape(n, heads * _HF)


def kernel(x, params, edge_index):
    src = edge_index[0]
    dst = edge_index[1]
    n = x.shape[0]

    h, s0 = _matmul_bias(x, params['enc_W0'], params['enc_b0'])
    h = _bn_leaky(h, s0, params['bn_g0'], params['bn_b0'])
    h, s1 = _matmul_bias(h, params['enc_W1'], params['enc_b1'])
    h = _bn_leaky(h, s1, params['bn_g1'], params['bn_b1'])

    for i in range(5):
        W = params['gat_W'][i]
        al = params['gat_al'][i]
        ar = params['gat_ar'][i]
        heads = al.shape[0]
        din = W.shape[0]
        wl = jnp.concatenate(
            [W[:, hh * _HF:(hh + 1) * _HF] @ al[hh] for hh in range(heads)], 0
        ).reshape(heads, din).T
        wr = jnp.concatenate(
            [W[:, hh * _HF:(hh + 1) * _HF] @ ar[hh] for hh in range(heads)], 0
        ).reshape(heads, din).T
        pad = jnp.zeros((din, 8 - 2 * heads), jnp.float32)
        wcat = jnp.concatenate([W, wl, wr, pad], axis=1)
        bcat = jnp.concatenate(
            [params['gat_b'][i], jnp.zeros((8,), jnp.float32)], 0
        )
        out, _ = _matmul_bias(h, wcat, bcat)
        feat = out[:, : heads * _HF]
        el = out[:, heads * _HF: heads * _HF + heads]
        er = out[:, heads * _HF + heads: heads * _HF + 2 * heads]
        h = _gat_edge(feat, el, er, src, dst, n, heads, params['gat_b'][i])

    out, _ = _matmul_bias(h, params['fc_W'], params['fc_b'])
    return out


# speedup vs baseline: 1.1471x; 1.1471x over previous
import functools
import jax
import jax.numpy as jnp
from jax import lax
from jax.experimental import pallas as pl
from jax.experimental.pallas import tpu as pltpu
from jax.experimental.pallas import tpu_sc as plsc

_N = 50000
_HF = 64
_ROWS = 2000

_E = 800000
_NW = 32
_NP = 50176
_RPT = _NP // _NW
_EPT = _E // _NW
_ECH = 1000
_NCH = _EPT // _ECH


def _take16(x, idx):
    dn = lax.GatherDimensionNumbers(
        offset_dims=(), collapsed_slice_dims=(0,), start_index_map=(0,))
    return lax.gather(x, idx[:, None], dn, (1,),
                      mode=lax.GatherScatterMode.PROMISE_IN_BOUNDS)


def _lanes():
    lane = lax.iota(jnp.int32, 16)
    mask4 = jnp.where(lane < 4, 1.0, 0.0)
    oh4 = jnp.where(lane == 4, 1.0, 0.0)
    return mask4, oh4


def _wid():
    return lax.axis_index("s") * 2 + lax.axis_index("c")


def _sc_mesh():
    return plsc.VectorSubcoreMesh(core_axis_name="c", subcore_axis_name="s")


def _mm_body(x_ref, w_ref, b_ref, o_ref, s_ref):
    i = pl.program_id(0)
    h = jnp.dot(x_ref[...], w_ref[...], preferred_element_type=jnp.float32)
    h = h + b_ref[...]
    o_ref[...] = h

    @pl.when(i == 0)
    def _():
        s_ref[...] = jnp.zeros_like(s_ref)

    s_ref[0:1, :] += jnp.sum(h, axis=0, keepdims=True)
    s_ref[1:2, :] += jnp.sum(h * h, axis=0, keepdims=True)


def _matmul_bias(x, w, b):
    n, k = x.shape
    m = w.shape[1]
    grid = n // _ROWS
    out, stats = pl.pallas_call(
        _mm_body,
        grid=(grid,),
        in_specs=[
            pl.BlockSpec((_ROWS, k), lambda i: (i, 0)),
            pl.BlockSpec((k, m), lambda i: (0, 0)),
            pl.BlockSpec((1, m), lambda i: (0, 0)),
        ],
        out_specs=[
            pl.BlockSpec((_ROWS, m), lambda i: (i, 0)),
            pl.BlockSpec((2, m), lambda i: (0, 0)),
        ],
        out_shape=[
            jax.ShapeDtypeStruct((n, m), jnp.float32),
            jax.ShapeDtypeStruct((2, m), jnp.float32),
        ],
    )(x, w, b.reshape(1, -1))
    return out, stats


def _bn_body(h_ref, s_ref, g_ref, b_ref, o_ref):
    s = s_ref[...]
    mean = s[0:1, :] / _N
    var = s[1:2, :] / _N - mean * mean
    y = (h_ref[...] - mean) * jax.lax.rsqrt(var + 1e-5) * g_ref[...] + b_ref[...]
    o_ref[...] = jnp.where(y > 0, y, 0.01 * y)


def _bn_leaky(h, stats, g, b):
    n, m = h.shape
    return pl.pallas_call(
        _bn_body,
        grid=(n // _ROWS,),
        in_specs=[
            pl.BlockSpec((_ROWS, m), lambda i: (i, 0)),
            pl.BlockSpec((2, m), lambda i: (0, 0)),
            pl.BlockSpec((1, m), lambda i: (0, 0)),
            pl.BlockSpec((1, m), lambda i: (0, 0)),
        ],
        out_specs=pl.BlockSpec((_ROWS, m), lambda i: (i, 0)),
        out_shape=jax.ShapeDtypeStruct((n, m), jnp.float32),
    )(h, stats, g.reshape(1, -1), b.reshape(1, -1))


def _zero_own_range(shared, zbuf, sid):
    zb = zbuf.shape[0]
    rows = _NP // 16

    def zrow(j, _):
        zbuf[j] = jnp.zeros((16,), jnp.float32)
        return 0

    lax.fori_loop(0, zb, zrow, 0)

    def zchunk(k, _):
        pltpu.sync_copy(zbuf, shared.at[pl.ds(sid * rows + k * zb, zb)])
        return 0

    lax.fori_loop(0, rows // zb, zchunk, 0)


@functools.partial(
    pl.kernel,
    mesh=_sc_mesh(),
    compiler_params=pltpu.CompilerParams(use_tc_tiling_on_sc=False, needs_layout_passes=False),
    out_type=jax.ShapeDtypeStruct((2, _NP, 16), jnp.float32),
    scratch_types=[
        pltpu.VMEM((_ECH,), jnp.int32),
        pltpu.VMEM((_ECH,), jnp.int32),
        pltpu.VMEM((_ECH, 16), jnp.float32),
        pltpu.VMEM((_ECH, 16), jnp.float32),
        pltpu.VMEM((_ECH, 16), jnp.float32),
        pltpu.VMEM((784, 16), jnp.float32),
        pltpu.VMEM_SHARED((_NP, 16), jnp.float32),
    ],
)
def _sc_sums(src_h, dst_h, ela_h, erb_h, part_h, srcv, dstv, gsv, gdv, wv,
             zbuf, sums_sh):
    cid = lax.axis_index("c")
    sid = lax.axis_index("s")
    wid = _wid()
    mask4, oh4 = _lanes()
    _zero_own_range(sums_sh, zbuf, sid)
    plsc.subcore_barrier()

    def chunk(k, _):
        base = wid * _EPT + k * _ECH
        pltpu.sync_copy(src_h.at[pl.ds(base, _ECH)], srcv)
        pltpu.sync_copy(dst_h.at[pl.ds(base, _ECH)], dstv)
        pltpu.sync_copy(ela_h.at[srcv], gsv)
        pltpu.sync_copy(erb_h.at[dstv], gdv)

        def edge(j, _):
            z = gsv[j] + gdv[j]
            e = jnp.where(z > 0, z, 0.2 * z)
            wv[j] = e + oh4
            return 0

        lax.fori_loop(0, _ECH, edge, 0)
        pltpu.sync_copy(wv, sums_sh.at[dstv], add=True)
        return 0

    lax.fori_loop(0, _NCH, chunk, 0)
    plsc.subcore_barrier()
    rows = _NP // 16
    pltpu.sync_copy(sums_sh.at[pl.ds(sid * rows, rows)],
                    part_h.at[cid, pl.ds(sid * rows, rows)])


@functools.partial(
    pl.kernel,
    mesh=_sc_mesh(),
    compiler_params=pltpu.CompilerParams(use_tc_tiling_on_sc=False, needs_layout_passes=False),
    out_type=jax.ShapeDtypeStruct((_NP, 16), jnp.float32),
    scratch_types=[
        pltpu.VMEM((_RPT, 16), jnp.float32),
        pltpu.VMEM((_RPT, 16), jnp.float32),
        pltpu.VMEM((_RPT, 16), jnp.float32),
    ],
)
def _sc_mean(part_h, mean_h, b0, b1, bm):
    wid = _wid()
    mask4, oh4 = _lanes()
    base = wid * _RPT
    pltpu.sync_copy(part_h.at[0, pl.ds(base, _RPT)], b0)
    pltpu.sync_copy(part_h.at[1, pl.ds(base, _RPT)], b1)

    def row(j, _):
        r = b0[j] + b1[j]
        deg = jnp.sum(r * oh4)
        bm[j] = (r * mask4) / jnp.maximum(deg, 1.0)
        return 0

    lax.fori_loop(0, _RPT, row, 0)
    pltpu.sync_copy(bm, mean_h.at[pl.ds(base, _RPT)])


@functools.partial(
    pl.kernel,
    mesh=_sc_mesh(),
    compiler_params=pltpu.CompilerParams(use_tc_tiling_on_sc=False, needs_layout_passes=False),
    out_type=jax.ShapeDtypeStruct((2, _NP, 16), jnp.float32),
    scratch_types=[
        pltpu.VMEM((_ECH,), jnp.int32),
        pltpu.VMEM((_ECH,), jnp.int32),
        pltpu.VMEM((_ECH, 16), jnp.float32),
        pltpu.VMEM((_ECH, 16), jnp.float32),
        pltpu.VMEM((_ECH, 16), jnp.float32),
        pltpu.VMEM((_ECH, 16), jnp.float32),
        pltpu.VMEM((784, 16), jnp.float32),
        pltpu.VMEM_SHARED((_NP, 16), jnp.float32),
    ],
)
def _sc_den(src_h, dst_h, ela_h, erb_h, mean_h, part_h, srcv, dstv, gsv, gdv,
            mdv, wv, zbuf, den_sh):
    cid = lax.axis_index("c")
    sid = lax.axis_index("s")
    wid = _wid()
    mask4, oh4 = _lanes()
    _zero_own_range(den_sh, zbuf, sid)
    plsc.subcore_barrier()

    def chunk(k, _):
        base = wid * _EPT + k * _ECH
        pltpu.sync_copy(src_h.at[pl.ds(base, _ECH)], srcv)
        pltpu.sync_copy(dst_h.at[pl.ds(base, _ECH)], dstv)
        pltpu.sync_copy(ela_h.at[srcv], gsv)
        pltpu.sync_copy(erb_h.at[dstv], gdv)
        pltpu.sync_copy(mean_h.at[dstv], mdv)

        def edge(j, _):
            z = gsv[j] + gdv[j]
            e = jnp.where(z > 0, z, 0.2 * z)
            wv[j] = jnp.exp((e - mdv[j]) * mask4) * mask4
            return 0

        lax.fori_loop(0, _ECH, edge, 0)
        pltpu.sync_copy(wv, den_sh.at[dstv], add=True)
        return 0

    lax.fori_loop(0, _NCH, chunk, 0)
    plsc.subcore_barrier()
    rows = _NP // 16
    pltpu.sync_copy(den_sh.at[pl.ds(sid * rows, rows)],
                    part_h.at[cid, pl.ds(sid * rows, rows)])


@functools.partial(
    pl.kernel,
    mesh=_sc_mesh(),
    compiler_params=pltpu.CompilerParams(use_tc_tiling_on_sc=False, needs_layout_passes=False),
    out_type=jax.ShapeDtypeStruct((_NP, 16), jnp.float32),
    scratch_types=[
        pltpu.VMEM((_RPT, 16), jnp.float32),
        pltpu.VMEM((_RPT, 16), jnp.float32),
        pltpu.VMEM((_RPT, 16), jnp.float32),
        pltpu.VMEM((_RPT, 16), jnp.float32),
    ],
)
def _sc_md(part_h, mean_h, md_h, b0, b1, bm, bo):
    wid = _wid()
    mask4, oh4 = _lanes()
    base = wid * _RPT
    pltpu.sync_copy(part_h.at[0, pl.ds(base, _RPT)], b0)
    pltpu.sync_copy(part_h.at[1, pl.ds(base, _RPT)], b1)
    pltpu.sync_copy(mean_h.at[pl.ds(base, _RPT)], bm)

    def row(j, _):
        den = b0[j] + b1[j]
        inv = mask4 / (den + 1e-9)
        lane = lax.iota(jnp.int32, 16)
        ivr = _take16(inv, (lane + 12) & 15)
        bo[j] = bm[j] + ivr
        return 0

    lax.fori_loop(0, _RPT, row, 0)
    pltpu.sync_copy(bo, md_h.at[pl.ds(base, _RPT)])


@functools.partial(
    pl.kernel,
    mesh=_sc_mesh(),
    compiler_params=pltpu.CompilerParams(use_tc_tiling_on_sc=False, needs_layout_passes=False),
    out_type=jax.ShapeDtypeStruct((_E, 16), jnp.float32),
    scratch_types=[
        pltpu.VMEM((_ECH,), jnp.int32),
        pltpu.VMEM((_ECH,), jnp.int32),
        pltpu.VMEM((_ECH, 16), jnp.float32),
        pltpu.VMEM((_ECH, 16), jnp.float32),
        pltpu.VMEM((_ECH, 16), jnp.float32),
        pltpu.VMEM((_ECH, 16), jnp.float32),
    ],
)
def _sc_alpha(src_h, dst_h, ela_h, erb_h, md_h, alpha_h, srcv, dstv, gsv,
              gdv, mdv, wv):
    wid = _wid()
    mask4, oh4 = _lanes()

    def chunk(k, _):
        base = wid * _EPT + k * _ECH
        pltpu.sync_copy(src_h.at[pl.ds(base, _ECH)], srcv)
        pltpu.sync_copy(dst_h.at[pl.ds(base, _ECH)], dstv)
        pltpu.sync_copy(ela_h.at[srcv], gsv)
        pltpu.sync_copy(erb_h.at[dstv], gdv)
        pltpu.sync_copy(md_h.at[dstv], mdv)

        def edge(j, _):
            z = gsv[j] + gdv[j]
            e = jnp.where(z > 0, z, 0.2 * z)
            md = mdv[j]
            x1 = jnp.exp((e - md) * mask4)
            lane = lax.iota(jnp.int32, 16)
            iv = _take16(md, (lane + 4) & 15)
            wv[j] = x1 * iv * mask4
            return 0

        lax.fori_loop(0, _ECH, edge, 0)
        pltpu.sync_copy(wv, alpha_h.at[pl.ds(base, _ECH)])
        return 0

    lax.fori_loop(0, _NCH, chunk, 0)


def _edge_alpha_sc(el, er, src, dst, heads):
    pad = jnp.zeros((_NP, 16), jnp.float32)
    ela = pad.at[:_N, :heads].set(el)
    erb = pad.at[:_N, :heads].set(er)
    part = _sc_sums(src, dst, ela, erb)
    mean = _sc_mean(part)
    part2 = _sc_den(src, dst, ela, erb, mean)
    md = _sc_md(part2, mean)
    alpha = _sc_alpha(src, dst, ela, erb, md)
    return alpha[:, :heads]


def _gat_edge(feat, el, er, src, dst, n, heads, bias):
    alpha = _edge_alpha_sc(el, er, src, dst, heads)
    msg = feat.reshape(n, heads, _HF)[src] * alpha[:, :, None]
    rst = jax.ops.segment_sum(msg, dst, num_segments=n)
    out = rst + bias.reshape(1, heads, _HF)
    out = jnp.where(out > 0, out, 0.01 * out)
    return out.reshape(n, heads * _HF)


def kernel(x, params, edge_index):
    src = edge_index[0]
    dst = edge_index[1]
    n = x.shape[0]

    h, s0 = _matmul_bias(x, params['enc_W0'], params['enc_b0'])
    h = _bn_leaky(h, s0, params['bn_g0'], params['bn_b0'])
    h, s1 = _matmul_bias(h, params['enc_W1'], params['enc_b1'])
    h = _bn_leaky(h, s1, params['bn_g1'], params['bn_b1'])

    for i in range(5):
        W = params['gat_W'][i]
        al = params['gat_al'][i]
        ar = params['gat_ar'][i]
        heads = al.shape[0]
        din = W.shape[0]
        wl = jnp.concatenate(
            [W[:, hh * _HF:(hh + 1) * _HF] @ al[hh] for hh in range(heads)], 0
        ).reshape(heads, din).T
        wr = jnp.concatenate(
            [W[:, hh * _HF:(hh + 1) * _HF] @ ar[hh] for hh in range(heads)], 0
        ).reshape(heads, din).T
        pad = jnp.zeros((din, 8 - 2 * heads), jnp.float32)
        wcat = jnp.concatenate([W, wl, wr, pad], axis=1)
        bcat = jnp.concatenate(
            [params['gat_b'][i], jnp.zeros((8,), jnp.float32)], 0
        )
        out, _ = _matmul_bias(h, wcat, bcat)
        feat = out[:, : heads * _HF]
        el = out[:, heads * _HF: heads * _HF + heads]
        er = out[:, heads * _HF + heads: heads * _HF + 2 * heads]
        h = _gat_edge(feat, el, er, src, dst, n, heads, params['gat_b'][i])

    out, _ = _matmul_bias(h, params['fc_W'], params['fc_b'])
    return out
